# baseline (speedup 1.0000x reference)
import functools

import jax
import jax.numpy as jnp
from jax import lax
from jax.experimental import pallas as pl
from jax.experimental.pallas import tpu as pltpu
from jax.experimental.pallas import tpu_sc as plsc

CHUNK = 80
NSLOT = 4
LANES = 16
ZROWS = 125


def _body(nat, n_chunks, d_core, n_cores, n_sub,
          edge_hbm, src_hbm, dst_hbm, out_hbm,
          acc, rows0, rows1, rows2, rows3, neg0, neg1, neg2, neg3,
          idx0, idx1, idx2, idx3, zbuf,
          sem_l0, sem_l1, sem_l2, sem_l3, sem_s0, sem_s1, sem_s2, sem_s3):
    c = lax.axis_index("c")
    s = lax.axis_index("s")
    rows_per_sub = nat // n_sub
    col0 = c * d_core

    rows_b = (rows0, rows1, rows2, rows3)
    neg_b = (neg0, neg1, neg2, neg3)
    idx_b = (idx0, idx1, idx2, idx3)
    sem_l = (sem_l0, sem_l1, sem_l2, sem_l3)
    sem_s = (sem_s0, sem_s1, sem_s2, sem_s3)

    cnt = n_chunks // n_sub
    start = s * cnt

    def load_args(gi, b):
        ch = start + gi
        return (
            (src_hbm.at[ch], idx_b[b].at[0]),
            (dst_hbm.at[ch], idx_b[b].at[1]),
            (edge_hbm.at[pl.ds(ch * CHUNK, CHUNK),
                         pl.ds(col0, d_core)], rows_b[b]),
        )

    def start_loads(gi, b):
        for src, dst in load_args(gi, b):
            pltpu.async_copy(src, dst, sem_l[b])

    def wait_loads(gi, b):
        for src, dst in load_args(gi, b):
            pltpu.make_async_copy(src, dst, sem_l[b]).wait()

    def drain_scatters(b):
        pltpu.make_async_copy(rows_b[b], acc.at[idx_b[b].at[0]], sem_s[b]).wait()
        pltpu.make_async_copy(neg_b[b], acc.at[idx_b[b].at[1]], sem_s[b]).wait()

    start_loads(0, 0)
    start_loads(1, 1)

    ncg = d_core // LANES

    def zero_row(i, _):
        for k in range(ncg):
            zbuf[i, pl.ds(k * LANES, LANES)] = jnp.zeros((LANES,), jnp.float32)
        return 0

    lax.fori_loop(0, ZROWS, zero_row, 0)
    for b in range(rows_per_sub // ZROWS):
        base = s * rows_per_sub + b * ZROWS
        pltpu.sync_copy(zbuf, acc.at[pl.ds(base, ZROWS)])
    plsc.subcore_barrier()

    def loop_body(go, _):
        for b in range(NSLOT):
            gi = go * NSLOT + b
            sb = (b + NSLOT - 1) % NSLOT
            db = (b + NSLOT - 2) % NSLOT

            @pl.when((gi >= 2) & (gi <= cnt + 1))
            def _():
                drain_scatters(db)

            @pl.when(gi + 2 < cnt)
            def _():
                start_loads(gi + 2, db)

            @pl.when((gi >= 1) & (gi <= cnt))
            def _():
                pltpu.async_copy(
                    rows_b[sb], acc.at[idx_b[sb].at[0]], sem_s[sb], add=True)
                pltpu.async_copy(
                    neg_b[sb], acc.at[idx_b[sb].at[1]], sem_s[sb], add=True)

            @pl.when(gi < cnt)
            def _():
                wait_loads(gi, b)

                def neg_rows(i, _):
                    for r in range(5):
                        for k in range(ncg):
                            sl = pl.ds(k * LANES, LANES)
                            neg_b[b][i * 5 + r, sl] = -rows_b[b][i * 5 + r, sl]
                    return 0

                lax.fori_loop(0, CHUNK // 5, neg_rows, 0)

        return 0

    lax.fori_loop(0, (cnt + 2 + NSLOT - 1) // NSLOT + 1, loop_body, 0)
    plsc.subcore_barrier()

    row0 = s * rows_per_sub
    pltpu.sync_copy(acc.at[pl.ds(row0, rows_per_sub)],
                    out_hbm.at[pl.ds(row0, rows_per_sub), pl.ds(col0, d_core)])


def kernel(edge_attr, edge_src, edge_dst, species):
    nat = species.shape[0]
    n_edges, d_feat = edge_attr.shape
    info = plsc.get_sparse_core_info()
    n_cores, n_sub = info.num_cores, info.num_subcores
    d_core = d_feat // n_cores
    n_chunks = n_edges // CHUNK

    src2d = edge_src.reshape(n_chunks, CHUNK)
    dst2d = edge_dst.reshape(n_chunks, CHUNK)

    mesh = plsc.VectorSubcoreMesh(core_axis_name="c", subcore_axis_name="s")
    body = functools.partial(_body, nat, n_chunks, d_core, n_cores, n_sub)
    k = pl.kernel(
        body,
        out_type=jax.ShapeDtypeStruct((nat, d_feat), jnp.float32),
        mesh=mesh,
        scratch_types=[
            pltpu.VMEM_SHARED((nat, d_core), jnp.float32),
            pltpu.VMEM((CHUNK, d_core), jnp.float32),
            pltpu.VMEM((CHUNK, d_core), jnp.float32),
            pltpu.VMEM((CHUNK, d_core), jnp.float32),
            pltpu.VMEM((CHUNK, d_core), jnp.float32),
            pltpu.VMEM((CHUNK, d_core), jnp.float32),
            pltpu.VMEM((CHUNK, d_core), jnp.float32),
            pltpu.VMEM((CHUNK, d_core), jnp.float32),
            pltpu.VMEM((CHUNK, d_core), jnp.float32),
            pltpu.VMEM((2, CHUNK), jnp.int32),
            pltpu.VMEM((2, CHUNK), jnp.int32),
            pltpu.VMEM((2, CHUNK), jnp.int32),
            pltpu.VMEM((2, CHUNK), jnp.int32),
            pltpu.VMEM((ZROWS, d_core), jnp.float32),
            pltpu.SemaphoreType.DMA,
            pltpu.SemaphoreType.DMA,
            pltpu.SemaphoreType.DMA,
            pltpu.SemaphoreType.DMA,
            pltpu.SemaphoreType.DMA,
            pltpu.SemaphoreType.DMA,
            pltpu.SemaphoreType.DMA,
            pltpu.SemaphoreType.DMA,
        ],
        compiler_params=pltpu.CompilerParams(use_tc_tiling_on_sc=False),
    )
    return k(edge_attr, src2d, dst2d)


# speedup vs baseline: 1.1243x; 1.1243x over previous
import functools

import jax
import jax.numpy as jnp
from jax import lax
from jax.experimental import pallas as pl
from jax.experimental.pallas import tpu as pltpu
from jax.experimental.pallas import tpu_sc as plsc

CHUNK = 128
NSLOT = 3
LANES = 16
FROWS = 25
ZROWS = 125


def _body(nat, n_chunks, d_core, n_cores, n_sub,
          edge_hbm, idx_hbm, out_hbm,
          acc_src, acc_dst, rows0, rows1, rows2, idx0, idx1, idx2,
          zbuf, fa0, fb0, fa1, fb1,
          sem_l0, sem_l1, sem_l2, sem_s0, sem_s1, sem_s2, sem_f0, sem_f1,
          sem_o):
    c = lax.axis_index("c")
    s = lax.axis_index("s")
    rows_per_sub = nat // n_sub
    col0 = c * d_core

    rows_b = (rows0, rows1, rows2)
    idx_b = (idx0, idx1, idx2)
    sem_l = (sem_l0, sem_l1, sem_l2)
    sem_s = (sem_s0, sem_s1, sem_s2)
    fa_p = (fa0, fa1)
    fb_p = (fb0, fb1)
    sem_f = (sem_f0, sem_f1)

    n_base = n_chunks // n_sub
    n_rem = n_chunks % n_sub
    cnt = n_base + jnp.where(s < n_rem, 1, 0)
    start = s * n_base + jnp.minimum(s, n_rem)
    t_static = n_base + (1 if n_rem else 0)

    def load_args(gi, b):
        ch = start + gi
        return (
            (idx_hbm.at[ch], idx_b[b]),
            (edge_hbm.at[pl.ds(ch * CHUNK, CHUNK),
                         pl.ds(col0, d_core)], rows_b[b]),
        )

    def start_loads(gi, b):
        for src, dst in load_args(gi, b):
            pltpu.async_copy(src, dst, sem_l[b])

    def wait_loads(gi, b):
        for src, dst in load_args(gi, b):
            pltpu.make_async_copy(src, dst, sem_l[b]).wait()

    def drain_scatters(b):
        pltpu.make_async_copy(rows_b[b], acc_src.at[idx_b[b].at[0]], sem_s[b]).wait()
        pltpu.make_async_copy(rows_b[b], acc_dst.at[idx_b[b].at[1]], sem_s[b]).wait()

    start_loads(0, 0)
    start_loads(1, 1)

    ncg = d_core // LANES

    def zero_row(i, _):
        for k in range(ncg):
            zbuf[i, pl.ds(k * LANES, LANES)] = jnp.zeros((LANES,), jnp.float32)
        return 0

    lax.fori_loop(0, ZROWS, zero_row, 0)
    for b in range(rows_per_sub // ZROWS):
        base = s * rows_per_sub + b * ZROWS
        pltpu.sync_copy(zbuf, acc_src.at[pl.ds(base, ZROWS)])
        pltpu.sync_copy(zbuf, acc_dst.at[pl.ds(base, ZROWS)])
    plsc.subcore_barrier()

    def loop_body(go, _):
        for b in range(NSLOT):
            gi = go * NSLOT + b
            pb = (b + NSLOT - 1) % NSLOT

            @pl.when((gi >= 1) & (gi <= cnt))
            def _():
                drain_scatters(pb)

            @pl.when(gi + 2 < cnt)
            def _():
                start_loads(gi + 2, pb)

            @pl.when(gi < cnt)
            def _():
                wait_loads(gi, b)
                pltpu.async_copy(
                    rows_b[b], acc_src.at[idx_b[b].at[0]], sem_s[b], add=True)
                pltpu.async_copy(
                    rows_b[b], acc_dst.at[idx_b[b].at[1]], sem_s[b], add=True)

        return 0

    lax.fori_loop(0, (t_static + NSLOT) // NSLOT, loop_body, 0)
    plsc.subcore_barrier()

    n_fb = rows_per_sub // FROWS

    def fbase(i):
        return s * rows_per_sub + i * FROWS

    def start_floads(i, p):
        pltpu.async_copy(acc_src.at[pl.ds(fbase(i), FROWS)], fa_p[p], sem_f[p])
        pltpu.async_copy(acc_dst.at[pl.ds(fbase(i), FROWS)], fb_p[p], sem_f[p])

    def wait_floads(i, p):
        pltpu.make_async_copy(acc_src.at[pl.ds(fbase(i), FROWS)], fa_p[p], sem_f[p]).wait()
        pltpu.make_async_copy(acc_dst.at[pl.ds(fbase(i), FROWS)], fb_p[p], sem_f[p]).wait()

    store_descs = []
    start_floads(0, 0)
    for i in range(n_fb):
        p = i % 2
        if i + 1 < n_fb:
            if i >= 1:
                store_descs[i - 1].wait()
            start_floads(i + 1, 1 - p)
        wait_floads(i, p)

        def sub_row(r, _):
            for rr in range(5):
                for k in range(ncg):
                    sl = pl.ds(k * LANES, LANES)
                    fa_p[p][r * 5 + rr, sl] = (
                        fa_p[p][r * 5 + rr, sl] - fb_p[p][r * 5 + rr, sl])
            return 0

        lax.fori_loop(0, FROWS // 5, sub_row, 0)
        store_descs.append(pltpu.async_copy(
            fa_p[p], out_hbm.at[pl.ds(fbase(i), FROWS), pl.ds(col0, d_core)],
            sem_o))
    store_descs[n_fb - 2].wait()
    store_descs[n_fb - 1].wait()


def kernel(edge_attr, edge_src, edge_dst, species):
    nat = species.shape[0]
    n_edges, d_feat = edge_attr.shape
    info = plsc.get_sparse_core_info()
    n_cores, n_sub = info.num_cores, info.num_subcores
    d_core = d_feat // n_cores
    n_chunks = n_edges // CHUNK

    idx2d = jnp.stack(
        [edge_src.reshape(n_chunks, CHUNK), edge_dst.reshape(n_chunks, CHUNK)],
        axis=1)

    mesh = plsc.VectorSubcoreMesh(core_axis_name="c", subcore_axis_name="s")
    body = functools.partial(_body, nat, n_chunks, d_core, n_cores, n_sub)
    k = pl.kernel(
        body,
        out_type=jax.ShapeDtypeStruct((nat, d_feat), jnp.float32),
        mesh=mesh,
        scratch_types=[
            pltpu.VMEM_SHARED((nat, d_core), jnp.float32),
            pltpu.VMEM_SHARED((nat, d_core), jnp.float32),
            pltpu.VMEM((CHUNK, d_core), jnp.float32),
            pltpu.VMEM((CHUNK, d_core), jnp.float32),
            pltpu.VMEM((CHUNK, d_core), jnp.float32),
            pltpu.VMEM((2, CHUNK), jnp.int32),
            pltpu.VMEM((2, CHUNK), jnp.int32),
            pltpu.VMEM((2, CHUNK), jnp.int32),
            pltpu.VMEM((ZROWS, d_core), jnp.float32),
            pltpu.VMEM((FROWS, d_core), jnp.float32),
            pltpu.VMEM((FROWS, d_core), jnp.float32),
            pltpu.VMEM((FROWS, d_core), jnp.float32),
            pltpu.VMEM((FROWS, d_core), jnp.float32),
            pltpu.SemaphoreType.DMA,
            pltpu.SemaphoreType.DMA,
            pltpu.SemaphoreType.DMA,
            pltpu.SemaphoreType.DMA,
            pltpu.SemaphoreType.DMA,
            pltpu.SemaphoreType.DMA,
            pltpu.SemaphoreType.DMA,
            pltpu.SemaphoreType.DMA,
            pltpu.SemaphoreType.DMA,
        ],
        compiler_params=pltpu.CompilerParams(use_tc_tiling_on_sc=False),
    )
    return k(edge_attr, idx2d)


# speedup vs baseline: 1.1431x; 1.0167x over previous
import functools

import jax
import jax.numpy as jnp
from jax import lax
from jax.experimental import pallas as pl
from jax.experimental.pallas import tpu as pltpu
from jax.experimental.pallas import tpu_sc as plsc

CHUNK = 80
NSLOT = 4
LANES = 16
FROWS = 125


def _body(nat, n_chunks, d_core, n_cores, n_sub,
          edge_hbm, src_hbm, dst_hbm, out_hbm,
          acc_src, acc_dst, rows0, rows1, rows2, rows3,
          idx0, idx1, idx2, idx3,
          fa, fb, sem_l0, sem_l1, sem_l2, sem_l3,
          sem_s0, sem_s1, sem_s2, sem_s3):
    c = lax.axis_index("c")
    s = lax.axis_index("s")
    rows_per_sub = nat // n_sub
    col0 = c * d_core

    rows_b = (rows0, rows1, rows2, rows3)
    idx_b = (idx0, idx1, idx2, idx3)
    sem_l = (sem_l0, sem_l1, sem_l2, sem_l3)
    sem_s = (sem_s0, sem_s1, sem_s2, sem_s3)

    cnt = n_chunks // n_sub
    start = s * cnt

    def load_args(gi, b):
        ch = start + gi
        return (
            (src_hbm.at[ch], idx_b[b].at[0]),
            (dst_hbm.at[ch], idx_b[b].at[1]),
            (edge_hbm.at[pl.ds(ch * CHUNK, CHUNK),
                         pl.ds(col0, d_core)], rows_b[b]),
        )

    def start_loads(gi, b):
        for src, dst in load_args(gi, b):
            pltpu.async_copy(src, dst, sem_l[b])

    def wait_loads(gi, b):
        for src, dst in load_args(gi, b):
            pltpu.make_async_copy(src, dst, sem_l[b]).wait()

    def drain_scatters(b):
        pltpu.make_async_copy(rows_b[b], acc_src.at[idx_b[b].at[0]], sem_s[b]).wait()
        pltpu.make_async_copy(rows_b[b], acc_dst.at[idx_b[b].at[1]], sem_s[b]).wait()

    start_loads(0, 0)
    start_loads(1, 1)

    frows = fa.shape[0]
    ncg = d_core // LANES

    def zero_row(i, _):
        for k in range(ncg):
            fa[i, pl.ds(k * LANES, LANES)] = jnp.zeros((LANES,), jnp.float32)
        return 0

    lax.fori_loop(0, frows, zero_row, 0)

    for b in range(rows_per_sub // frows):
        base = s * rows_per_sub + b * frows
        pltpu.sync_copy(fa, acc_src.at[pl.ds(base, frows)])
        pltpu.sync_copy(fa, acc_dst.at[pl.ds(base, frows)])
    plsc.subcore_barrier()

    def loop_body(go, _):
        for b in range(NSLOT):
            gi = go * NSLOT + b
            pb = (b + NSLOT - 2) % NSLOT

            @pl.when((gi >= 2) & (gi <= cnt + 1))
            def _():
                drain_scatters(pb)

            @pl.when(gi + 2 < cnt)
            def _():
                start_loads(gi + 2, pb)

            @pl.when(gi < cnt)
            def _():
                wait_loads(gi, b)
                pltpu.async_copy(
                    rows_b[b], acc_src.at[idx_b[b].at[0]], sem_s[b], add=True)
                pltpu.async_copy(
                    rows_b[b], acc_dst.at[idx_b[b].at[1]], sem_s[b], add=True)

        return 0

    lax.fori_loop(0, (cnt + 2 + NSLOT - 1) // NSLOT + 1, loop_body, 0)
    plsc.subcore_barrier()

    FR = 25
    n_fb = rows_per_sub // FR
    foff = (0, 50)
    sem_f = (sem_s0, sem_s1)
    sem_o = sem_s2

    def fbase(i):
        return s * rows_per_sub + i * FR

    def start_floads(i, p):
        pltpu.async_copy(acc_src.at[pl.ds(fbase(i), FR)],
                         fa.at[pl.ds(foff[p], FR)], sem_f[p])
        pltpu.async_copy(acc_dst.at[pl.ds(fbase(i), FR)],
                         fb.at[pl.ds(foff[p], FR)], sem_f[p])

    def wait_floads(i, p):
        pltpu.make_async_copy(acc_src.at[pl.ds(fbase(i), FR)],
                              fa.at[pl.ds(foff[p], FR)], sem_f[p]).wait()
        pltpu.make_async_copy(acc_dst.at[pl.ds(fbase(i), FR)],
                              fb.at[pl.ds(foff[p], FR)], sem_f[p]).wait()

    store_descs = []
    start_floads(0, 0)
    for i in range(n_fb):
        p = i % 2
        if i + 1 < n_fb:
            if i >= 1:
                store_descs[i - 1].wait()
            start_floads(i + 1, 1 - p)
        wait_floads(i, p)

        off = foff[p]

        def sub_row(r, _):
            for rr in range(5):
                for k in range(ncg):
                    sl = pl.ds(k * LANES, LANES)
                    ri = off + r * 5 + rr
                    fa[ri, sl] = fa[ri, sl] - fb[ri, sl]
            return 0

        lax.fori_loop(0, FR // 5, sub_row, 0)
        store_descs.append(pltpu.async_copy(
            fa.at[pl.ds(foff[p], FR)],
            out_hbm.at[pl.ds(fbase(i), FR), pl.ds(col0, d_core)], sem_o))
    store_descs[n_fb - 2].wait()
    store_descs[n_fb - 1].wait()


def kernel(edge_attr, edge_src, edge_dst, species):
    nat = species.shape[0]
    n_edges, d_feat = edge_attr.shape
    info = plsc.get_sparse_core_info()
    n_cores, n_sub = info.num_cores, info.num_subcores
    d_core = d_feat // n_cores
    n_chunks = n_edges // CHUNK

    src2d = edge_src.reshape(n_chunks, CHUNK)
    dst2d = edge_dst.reshape(n_chunks, CHUNK)

    mesh = plsc.VectorSubcoreMesh(core_axis_name="c", subcore_axis_name="s")
    body = functools.partial(_body, nat, n_chunks, d_core, n_cores, n_sub)
    k = pl.kernel(
        body,
        out_type=jax.ShapeDtypeStruct((nat, d_feat), jnp.float32),
        mesh=mesh,
        scratch_types=[
            pltpu.VMEM_SHARED((nat, d_core), jnp.float32),
            pltpu.VMEM_SHARED((nat, d_core), jnp.float32),
            pltpu.VMEM((CHUNK, d_core), jnp.float32),
            pltpu.VMEM((CHUNK, d_core), jnp.float32),
            pltpu.VMEM((CHUNK, d_core), jnp.float32),
            pltpu.VMEM((CHUNK, d_core), jnp.float32),
            pltpu.VMEM((2, CHUNK), jnp.int32),
            pltpu.VMEM((2, CHUNK), jnp.int32),
            pltpu.VMEM((2, CHUNK), jnp.int32),
            pltpu.VMEM((2, CHUNK), jnp.int32),
            pltpu.VMEM((FROWS, d_core), jnp.float32),
            pltpu.VMEM((FROWS, d_core), jnp.float32),
            pltpu.SemaphoreType.DMA,
            pltpu.SemaphoreType.DMA,
            pltpu.SemaphoreType.DMA,
            pltpu.SemaphoreType.DMA,
            pltpu.SemaphoreType.DMA,
            pltpu.SemaphoreType.DMA,
            pltpu.SemaphoreType.DMA,
            pltpu.SemaphoreType.DMA,
        ],
        compiler_params=pltpu.CompilerParams(use_tc_tiling_on_sc=False),
    )
    return k(edge_attr, src2d, dst2d)


# speedup vs baseline: 1.1665x; 1.0205x over previous
import functools

import jax
import jax.numpy as jnp
from jax import lax
from jax.experimental import pallas as pl
from jax.experimental.pallas import tpu as pltpu
from jax.experimental.pallas import tpu_sc as plsc

CHUNK = 80
NSLOT = 4
LANES = 16
FROWS = 125


def _body(nat, n_chunks, d_core, n_cores, n_sub,
          edge_hbm, src_hbm, dst_hbm, out_hbm,
          acc_src, acc_dst, rows0, rows1, rows2, rows3,
          idx0, idx1, idx2, idx3,
          fa, fb, sem_l0, sem_l1, sem_l2, sem_l3,
          sem_s0, sem_s1, sem_s2, sem_s3):
    c = lax.axis_index("c")
    s = lax.axis_index("s")
    rows_per_sub = nat // n_sub
    col0 = c * d_core

    rows_b = (rows0, rows1, rows2, rows3)
    idx_b = (idx0, idx1, idx2, idx3)
    sem_l = (sem_l0, sem_l1, sem_l2, sem_l3)
    sem_s = (sem_s0, sem_s1, sem_s2, sem_s3)

    cnt = n_chunks // n_sub
    start = s * cnt

    def load_args(gi, b):
        ch = start + gi
        return (
            (src_hbm.at[ch], idx_b[b].at[0]),
            (dst_hbm.at[ch], idx_b[b].at[1]),
            (edge_hbm.at[pl.ds(ch * CHUNK, CHUNK),
                         pl.ds(col0, d_core)], rows_b[b]),
        )

    def start_loads(gi, b):
        for src, dst in load_args(gi, b):
            pltpu.async_copy(src, dst, sem_l[b])

    def wait_loads(gi, b):
        for src, dst in load_args(gi, b):
            pltpu.make_async_copy(src, dst, sem_l[b]).wait()

    def drain_scatters(b):
        pltpu.make_async_copy(rows_b[b], acc_src.at[idx_b[b].at[0]], sem_s[b]).wait()
        pltpu.make_async_copy(rows_b[b], acc_dst.at[idx_b[b].at[1]], sem_s[b]).wait()

    start_loads(0, 0)
    start_loads(1, 1)

    frows = fa.shape[0]
    ncg = d_core // LANES

    def zero_row(i, _):
        for k in range(ncg):
            fa[i, pl.ds(k * LANES, LANES)] = jnp.zeros((LANES,), jnp.float32)
        return 0

    lax.fori_loop(0, frows, zero_row, 0)

    zdescs = []
    for b in range(rows_per_sub // frows):
        base = s * rows_per_sub + b * frows
        zdescs.append(pltpu.async_copy(fa, acc_src.at[pl.ds(base, frows)], sem_s0))
        zdescs.append(pltpu.async_copy(fa, acc_dst.at[pl.ds(base, frows)], sem_s1))
    for d in zdescs:
        d.wait()
    plsc.subcore_barrier()

    def loop_body(go, _):
        for b in range(NSLOT):
            gi = go * NSLOT + b
            pb = (b + NSLOT - 2) % NSLOT

            @pl.when((gi >= 2) & (gi <= cnt + 1))
            def _():
                drain_scatters(pb)

            @pl.when(gi + 2 < cnt)
            def _():
                start_loads(gi + 2, pb)

            @pl.when(gi < cnt)
            def _():
                wait_loads(gi, b)
                pltpu.async_copy(
                    rows_b[b], acc_src.at[idx_b[b].at[0]], sem_s[b], add=True)
                pltpu.async_copy(
                    rows_b[b], acc_dst.at[idx_b[b].at[1]], sem_s[b], add=True)

        return 0

    lax.fori_loop(0, (cnt + 2 + NSLOT - 1) // NSLOT + 1, loop_body, 0)
    plsc.subcore_barrier()

    for b in range(rows_per_sub // frows):
        row0 = s * rows_per_sub + b * frows
        pltpu.sync_copy(acc_src.at[pl.ds(row0, frows)], fa)
        pltpu.sync_copy(acc_dst.at[pl.ds(row0, frows)], fb)

        def sub_row(i, _):
            for r in range(5):
                for k in range(ncg):
                    sl = pl.ds(k * LANES, LANES)
                    ri = i * 5 + r
                    fa[ri, sl] = fa[ri, sl] - fb[ri, sl]
            return 0

        lax.fori_loop(0, frows // 5, sub_row, 0)

        pltpu.sync_copy(fa, out_hbm.at[pl.ds(row0, frows),
                                       pl.ds(col0, d_core)])


def kernel(edge_attr, edge_src, edge_dst, species):
    nat = species.shape[0]
    n_edges, d_feat = edge_attr.shape
    info = plsc.get_sparse_core_info()
    n_cores, n_sub = info.num_cores, info.num_subcores
    d_core = d_feat // n_cores
    n_chunks = n_edges // CHUNK

    src2d = edge_src.reshape(n_chunks, CHUNK)
    dst2d = edge_dst.reshape(n_chunks, CHUNK)

    mesh = plsc.VectorSubcoreMesh(core_axis_name="c", subcore_axis_name="s")
    body = functools.partial(_body, nat, n_chunks, d_core, n_cores, n_sub)
    k = pl.kernel(
        body,
        out_type=jax.ShapeDtypeStruct((nat, d_feat), jnp.float32),
        mesh=mesh,
        scratch_types=[
            pltpu.VMEM_SHARED((nat, d_core), jnp.float32),
            pltpu.VMEM_SHARED((nat, d_core), jnp.float32),
            pltpu.VMEM((CHUNK, d_core), jnp.float32),
            pltpu.VMEM((CHUNK, d_core), jnp.float32),
            pltpu.VMEM((CHUNK, d_core), jnp.float32),
            pltpu.VMEM((CHUNK, d_core), jnp.float32),
            pltpu.VMEM((2, CHUNK), jnp.int32),
            pltpu.VMEM((2, CHUNK), jnp.int32),
            pltpu.VMEM((2, CHUNK), jnp.int32),
            pltpu.VMEM((2, CHUNK), jnp.int32),
            pltpu.VMEM((FROWS, d_core), jnp.float32),
            pltpu.VMEM((FROWS, d_core), jnp.float32),
            pltpu.SemaphoreType.DMA,
            pltpu.SemaphoreType.DMA,
            pltpu.SemaphoreType.DMA,
            pltpu.SemaphoreType.DMA,
            pltpu.SemaphoreType.DMA,
            pltpu.SemaphoreType.DMA,
            pltpu.SemaphoreType.DMA,
            pltpu.SemaphoreType.DMA,
        ],
        compiler_params=pltpu.CompilerParams(use_tc_tiling_on_sc=False),
    )
    return k(edge_attr, src2d, dst2d)
